# baseline (speedup 1.0000x reference)
import functools

import jax
import jax.numpy as jnp
from jax.experimental import pallas as pl
from jax.experimental.pallas import tpu as pltpu

_N = 50000
_E = 64000
_H = 2
_C = 32
_REL_SRC = [0, 1, 0, 2, 1, 2, 3, 2]
_REL_DST = [1, 0, 2, 0, 2, 1, 2, 3]

_MLP_BLK = 2000


def _mlp_body(x_ref, u_ref, wz0, bz0, wz1, bz1, wz2, bz2,
              wv0, bv0, wv1, bv1, wv2, bv2, p_ref, z_ref, v_ref):
    x = x_ref[...]
    h = jax.nn.relu(jnp.dot(x, wz0[...], preferred_element_type=jnp.float32) + bz0[...])
    h = jax.nn.relu(jnp.dot(h, wz1[...], preferred_element_type=jnp.float32) + bz1[...])
    p = jax.nn.sigmoid(jnp.dot(h, wz2[...], preferred_element_type=jnp.float32) + bz2[...])
    g = jax.nn.relu(jnp.dot(x, wv0[...], preferred_element_type=jnp.float32) + bv0[...])
    g = jax.nn.relu(jnp.dot(g, wv1[...], preferred_element_type=jnp.float32) + bv1[...])
    v = jnp.dot(g, wv2[...], preferred_element_type=jnp.float32) + bv2[...]
    p_ref[...] = p
    z_ref[...] = (u_ref[...] < p).astype(jnp.float32)
    v_ref[...] = v


def _mlp_heads(gene, u8, Wz0, bz0, Wz1, bz1, Wz2, bz2, Wv0, bv0, Wv1, bv1, Wv2, bv2):
    nblk = _N // _MLP_BLK
    wz2p = jnp.pad(Wz2, ((0, 0), (0, 7)))
    wv2p = jnp.pad(Wv2, ((0, 0), (0, 7)))
    bz2p = jnp.pad(bz2, (0, 7)).reshape(1, 8)
    bv2p = jnp.pad(bv2, (0, 7)).reshape(1, 8)
    full = lambda shp: pl.BlockSpec(shp, lambda i: (0, 0))
    out_shape = [jax.ShapeDtypeStruct((_N, 8), jnp.float32)] * 3
    p8, z8, v8 = pl.pallas_call(
        _mlp_body,
        grid=(nblk,),
        in_specs=[
            pl.BlockSpec((_MLP_BLK, _C), lambda i: (i, 0)),
            pl.BlockSpec((_MLP_BLK, 8), lambda i: (i, 0)),
            full(Wz0.shape), full((1, 512)), full(Wz1.shape), full((1, 512)),
            full(wz2p.shape), full((1, 8)),
            full(Wv0.shape), full((1, 512)), full(Wv1.shape), full((1, 512)),
            full(wv2p.shape), full((1, 8)),
        ],
        out_specs=[pl.BlockSpec((_MLP_BLK, 8), lambda i: (i, 0))] * 3,
        out_shape=out_shape,
    )(gene, u8, Wz0, bz0.reshape(1, 512), Wz1, bz1.reshape(1, 512), wz2p, bz2p,
      Wv0, bv0.reshape(1, 512), Wv1, bv1.reshape(1, 512), wv2p, bv2p)
    return p8[:, :1], z8[:, :1], v8[:, :1]


def _gat(x_src, x_dst, ei, Ws, Wd, a_s, a_d, bias, n_dst):
    src, dst = ei[0], ei[1]
    xs = (x_src @ Ws).reshape(-1, _H, _C)
    xd = (x_dst @ Wd).reshape(-1, _H, _C)
    al_s = jnp.sum(xs * a_s[None], axis=-1)
    al_d = jnp.sum(xd * a_d[None], axis=-1)
    alpha = jax.nn.leaky_relu(al_s[src] + al_d[dst], negative_slope=0.2)
    amax = jax.ops.segment_max(alpha, dst, num_segments=n_dst)
    amax = jax.lax.stop_gradient(jnp.where(jnp.isfinite(amax), amax, 0.0))
    ex = jnp.exp(alpha - amax[dst])
    den = jax.ops.segment_sum(ex, dst, num_segments=n_dst)
    att = ex / (den[dst] + 1e-16)
    msg = (xs[src] * att[:, :, None]).reshape(-1, _H * _C)
    out = jax.ops.segment_sum(msg, dst, num_segments=n_dst).reshape(n_dst, _H, _C)
    return out.mean(axis=1) + bias


def kernel(x_tad, x_atac, x_gene, x_protein, ei0, ei1, ei2, ei3, ei4, ei5, ei6, ei7,
           W0_src, W0_dst, att0_src, att0_dst, b0, W_src, W_dst, att_src, att_dst, b,
           Wz0, bz0, Wz1, bz1, Wz2, bz2, Wv0, bv0, Wv1, bv1, Wv2, bv2):
    xs = [x_tad, x_atac, x_gene, x_protein]
    eis = [ei0, ei1, ei2, ei3, ei4, ei5, ei6, ei7]
    Ns = [x.shape[0] for x in xs]
    for l in range(4):
        if l == 0:
            Ws_l, Wd_l, as_l, ad_l, b_l = W0_src, W0_dst, att0_src, att0_dst, b0
        else:
            Ws_l, Wd_l, as_l, ad_l, b_l = (W_src[l - 1], W_dst[l - 1],
                                           att_src[l - 1], att_dst[l - 1], b[l - 1])
        new = [jnp.zeros((n, _C), dtype=xs[0].dtype) for n in Ns]
        for r in range(8):
            s, d = _REL_SRC[r], _REL_DST[r]
            new[d] = new[d] + _gat(xs[s], xs[d], eis[r], Ws_l[r], Wd_l[r],
                                   as_l[r], ad_l[r], b_l[r], Ns[d])
        xs = [jax.nn.relu(v) for v in new]

    gene = xs[2]
    u = jax.random.uniform(jax.random.key(1), (_N, 1), dtype=jnp.float32)
    u8 = jnp.broadcast_to(u, (_N, 8))
    p, z, v = _mlp_heads(gene, u8, Wz0, bz0, Wz1, bz1, Wz2, bz2,
                         Wv0, bv0, Wv1, bv1, Wv2, bv2)
    return (p, z, v)
